# baseline (speedup 1.0000x reference)
import functools

import jax
import jax.numpy as jnp
from jax import lax
from jax.experimental import pallas as pl
from jax.experimental.pallas import tpu as pltpu
from jax.experimental.pallas import tpu_sc as plsc

V = 1000000
D = 64
B = 16384
CTX = 20
K = 20

L = 16
NQ = D // L
NW = 32
EPW = B // NW
C = 16
NCHUNK = EPW // C
PADW = 128
GP = 24

_mesh = plsc.VectorSubcoreMesh(core_axis_name="c", subcore_axis_name="s")


@functools.partial(
    pl.kernel,
    out_type=jax.ShapeDtypeStruct((B // C, K + 1, C), jnp.float32),
    mesh=_mesh,
    scratch_types=[
        pltpu.VMEM((2, C, PADW), jnp.int32),
        pltpu.VMEM((EPW,), jnp.int32),
        pltpu.VMEM((2, C * GP, D), jnp.float32),
        pltpu.VMEM((2, C * GP, D), jnp.float32),
        pltpu.VMEM((2, C, D), jnp.float32),
        pltpu.VMEM((2, K + 1, L), jnp.float32),
        pltpu.SemaphoreType.DMA,
        pltpu.SemaphoreType.DMA,
        pltpu.SemaphoreType.DMA,
        pltpu.SemaphoreType.DMA,
    ],
    compiler_params=pltpu.CompilerParams(
        needs_layout_passes=False, use_tc_tiling_on_sc=False),
)
def _sc_scores(idx_hbm, tgt_hbm, win_hbm, wout_hbm, out_hbm,
               idx_v, tidx_v, crows_v, nrows_v, prows_v, scores_v,
               isem, gsem, ssem0, ssem1):
    wid = lax.axis_index("s") * 2 + lax.axis_index("c")
    base = wid * EPW
    lane = lax.iota(jnp.int32, L)

    pltpu.sync_copy(tgt_hbm.at[pl.ds(base, EPW)], tidx_v)

    def idx_desc(c, b):
        return pltpu.make_async_copy(
            idx_hbm.at[pl.ds(base + c * C, C), :], idx_v.at[b], isem)

    def gather_descs(c, b):
        ds = []
        for e in range(C):
            ds.append(pltpu.make_async_copy(
                win_hbm.at[idx_v.at[b, e, pl.ds(0, GP)]],
                crows_v.at[b, pl.ds(e * GP, GP), :], gsem))
        for e in range(C):
            ds.append(pltpu.make_async_copy(
                wout_hbm.at[idx_v.at[b, e, pl.ds(GP, GP)]],
                nrows_v.at[b, pl.ds(e * GP, GP), :], gsem))
        ds.append(pltpu.make_async_copy(
            wout_hbm.at[tidx_v.at[pl.ds(c * C, C)]],
            prows_v.at[b], gsem))
        return ds

    def score_desc(c, b):
        g = wid * NCHUNK + c
        sem = ssem0 if b == 0 else ssem1
        return pltpu.make_async_copy(scores_v.at[b], out_hbm.at[g], sem)

    idx_desc(0, 0).start()
    idx_desc(0, 0).wait()
    for d in gather_descs(0, 0):
        d.start()
    idx_desc(1, 1).start()

    @pl.loop(0, NCHUNK, step=2)
    def _pair(i):
        for b in range(2):
            chunk = i + b
            for d in gather_descs(chunk, b):
                d.wait()

            @pl.when(chunk + 1 < NCHUNK)
            def _prefetch():
                idx_desc(chunk + 1, 1 - b).wait()
                for d in gather_descs(chunk + 1, 1 - b):
                    d.start()

            @pl.when(chunk + 2 < NCHUNK)
            def _prefetch_idx():
                idx_desc(chunk + 2, b).start()

            @pl.when(chunk >= 2)
            def _drain():
                score_desc(chunk, b).wait()

            def _element(c, svec):
                h = [crows_v[b, c * GP, pl.ds(q * L, L)] for q in range(NQ)]
                for r in range(1, CTX):
                    for q in range(NQ):
                        h[q] = h[q] + crows_v[b, c * GP + r, pl.ds(q * L, L)]
                svec = list(svec)
                acc = h[0] * prows_v[b, c, pl.ds(0, L)]
                for q in range(1, NQ):
                    acc = acc + h[q] * prows_v[b, c, pl.ds(q * L, L)]
                s = plsc.cumsum(acc)[L - 1] * (-1.0 / CTX)
                svec[0] = jnp.where(lane == c, s, svec[0])
                for j in range(K):
                    acc = h[0] * nrows_v[b, c * GP + j, pl.ds(0, L)]
                    for q in range(1, NQ):
                        acc = acc + h[q] * nrows_v[b, c * GP + j, pl.ds(q * L, L)]
                    s = plsc.cumsum(acc)[L - 1] * (1.0 / CTX)
                    svec[j + 1] = jnp.where(lane == c, s, svec[j + 1])
                return tuple(svec)

            svec = lax.fori_loop(
                0, C, _element,
                tuple(jnp.zeros((L,), jnp.float32) for _ in range(K + 1)))
            for j in range(K + 1):
                scores_v[b, j, :] = svec[j]
            score_desc(chunk, b).start()

    for b in range(2):
        score_desc(b, b).wait()


def _loss_body(x_ref, o_ref):
    z = x_ref[...]
    sp = jnp.maximum(z, 0.0) + jnp.log1p(jnp.exp(-jnp.abs(z)))
    o_ref[0, 0] = jnp.sum(sp) * (1.0 / B)


_loss_call = pl.pallas_call(
    _loss_body,
    out_shape=jax.ShapeDtypeStruct((1, 1), jnp.float32),
    out_specs=pl.BlockSpec(memory_space=pltpu.SMEM),
)


def kernel(context, target, neg_targets, W_in, W_out):
    idx = jnp.concatenate(
        [jnp.pad(context.astype(jnp.int32), ((0, 0), (0, GP - CTX))),
         jnp.pad(neg_targets.astype(jnp.int32),
                 ((0, 0), (0, PADW - GP - K)))], axis=1)
    scores = _sc_scores(idx, target.astype(jnp.int32), W_in, W_out)
    loss = _loss_call(scores.reshape((K + 1) * B // 128, 128))
    return loss[0, 0]


# speedup vs baseline: 1.8704x; 1.8704x over previous
import functools

import jax
import jax.numpy as jnp
from jax import lax
from jax.experimental import pallas as pl
from jax.experimental.pallas import tpu as pltpu
from jax.experimental.pallas import tpu_sc as plsc

V = 1000000
D = 64
B = 16384
CTX = 20
K = 20

L = 16
NQ = D // L
NW = 32
EPW = B // NW
C = 8
NCHUNK = EPW // C
GW = 80
NGC = C * CTX // GW
NPAIR = (K + 2) // 2
ROWS = 2

_mesh = plsc.VectorSubcoreMesh(core_axis_name="c", subcore_axis_name="s")


@functools.partial(
    pl.kernel,
    out_type=jax.ShapeDtypeStruct((B // C * ROWS, 128), jnp.float32),
    mesh=_mesh,
    scratch_types=[
        pltpu.VMEM((EPW * CTX,), jnp.int32),
        pltpu.VMEM((EPW * K,), jnp.int32),
        pltpu.VMEM((EPW,), jnp.int32),
        pltpu.VMEM((2, C * CTX, D), jnp.float32),
        pltpu.VMEM((2, C * K, D), jnp.float32),
        pltpu.VMEM((2, C, D), jnp.float32),
        pltpu.VMEM((2, ROWS, 128), jnp.float32),
        pltpu.SemaphoreType.DMA,
        pltpu.SemaphoreType.DMA,
        pltpu.SemaphoreType.DMA,
    ],
    compiler_params=pltpu.CompilerParams(
        needs_layout_passes=False, use_tc_tiling_on_sc=False),
)
def _sc_scores(ctx_hbm, tgt_hbm, neg_hbm, win_hbm, wout_hbm, out_hbm,
               cidx_v, nidx_v, tidx_v, crows_v, nrows_v, prows_v, scores_v,
               gsem, ssem0, ssem1):
    wid = lax.axis_index("s") * 2 + lax.axis_index("c")
    lane = lax.iota(jnp.int32, L)

    pltpu.sync_copy(ctx_hbm.at[pl.ds(wid * EPW * CTX, EPW * CTX)], cidx_v)
    pltpu.sync_copy(neg_hbm.at[pl.ds(wid * EPW * K, EPW * K)], nidx_v)
    pltpu.sync_copy(tgt_hbm.at[pl.ds(wid * EPW, EPW)], tidx_v)

    neg_fill = jnp.full((L,), -100.0, jnp.float32)
    for b in range(2):
        for r in range(ROWS):
            for o in range(128 // L):
                scores_v[b, r, pl.ds(o * L, L)] = neg_fill

    def gather_descs(c, b):
        ds = []
        for j in range(NGC):
            ds.append(pltpu.make_async_copy(
                win_hbm.at[cidx_v.at[pl.ds(c * C * CTX + j * GW, GW)]],
                crows_v.at[b, pl.ds(j * GW, GW), :], gsem))
        for j in range(NGC):
            ds.append(pltpu.make_async_copy(
                wout_hbm.at[nidx_v.at[pl.ds(c * C * K + j * GW, GW)]],
                nrows_v.at[b, pl.ds(j * GW, GW), :], gsem))
        ds.append(pltpu.make_async_copy(
            wout_hbm.at[tidx_v.at[pl.ds(c * C, C)]],
            prows_v.at[b], gsem))
        return ds

    def score_desc(c, b):
        g = wid * NCHUNK + c
        sem = ssem0 if b == 0 else ssem1
        return pltpu.make_async_copy(
            scores_v.at[b], out_hbm.at[pl.ds(g * ROWS, ROWS), :], sem)

    for d in gather_descs(0, 0):
        d.start()

    @pl.loop(0, NCHUNK, step=2)
    def _pair(i):
        for b in range(2):
            chunk = i + b
            for d in gather_descs(chunk, b):
                d.wait()

            @pl.when(chunk + 1 < NCHUNK)
            def _prefetch():
                for d in gather_descs(chunk + 1, 1 - b):
                    d.start()

            @pl.when(chunk >= 2)
            def _drain():
                score_desc(chunk, b).wait()

            svec = [jnp.full((L,), -100.0, jnp.float32) for _ in range(NPAIR)]
            for c in range(C):
                h = [crows_v[b, c * CTX, pl.ds(q * L, L)] for q in range(NQ)]
                for r in range(1, CTX):
                    for q in range(NQ):
                        h[q] = h[q] + crows_v[b, c * CTX + r, pl.ds(q * L, L)]
                for j in range(K + 1):
                    if j == 0:
                        src, fac = prows_v, -1.0 / CTX
                        row = c
                    else:
                        src, fac = nrows_v, 1.0 / CTX
                        row = c * K + (j - 1)
                    acc = h[0] * src[b, row, pl.ds(0, L)]
                    for q in range(1, NQ):
                        acc = acc + h[q] * src[b, row, pl.ds(q * L, L)]
                    s = plsc.cumsum(acc)[L - 1] * fac
                    m, off = j // 2, (j % 2) * C
                    svec[m] = jnp.where(lane == c + off, s, svec[m])
            for m in range(NPAIR):
                scores_v[b, m // 8, pl.ds((m % 8) * L, L)] = svec[m]
            score_desc(chunk, b).start()

    for b in range(2):
        score_desc(b, b).wait()


def _loss_body(x_ref, o_ref):
    z = x_ref[...]
    sp = jnp.maximum(z, 0.0) + jnp.log1p(jnp.exp(-jnp.abs(z)))
    o_ref[0, 0] = jnp.sum(sp) * (1.0 / B)


_loss_call = pl.pallas_call(
    _loss_body,
    out_shape=jax.ShapeDtypeStruct((1, 1), jnp.float32),
    out_specs=pl.BlockSpec(memory_space=pltpu.SMEM),
)


def kernel(context, target, neg_targets, W_in, W_out):
    ctx_flat = context.astype(jnp.int32).reshape(-1)
    neg_flat = neg_targets.astype(jnp.int32).reshape(-1)
    tgt = target.astype(jnp.int32)
    scores = _sc_scores(ctx_flat, tgt, neg_flat, W_in, W_out)
    loss = _loss_call(scores)
    return loss[0, 0]
